# Initial kernel scaffold; baseline (speedup 1.0000x reference)
import jax
import jax.numpy as jnp
from jax.experimental import pallas as pl


def kernel(qk, v, rotations):
    raise NotImplementedError("write your pallas kernel here")


# speedup vs baseline: 2.6344x; 2.6344x over previous
import functools

import jax
import jax.numpy as jnp
from jax import lax
from jax.experimental import pallas as pl
from jax.experimental.pallas import tpu as pltpu
from jax.experimental.pallas import tpu_sc as plsc


def _hash_pos_body(qk_ref, v_ref, rot_ref, stg_ref, posg_ref, qv_ref):
    qk = qk_ref[0]
    rot = rot_ref[...]
    qv_ref[0] = jnp.concatenate([qk, v_ref[0]], axis=1)
    S = qk.shape[0]
    H = stg_ref.shape[1]
    HB = rot.shape[1] // H
    NB = 2 * HB
    b = pl.program_id(0)

    iota_nb = lax.broadcasted_iota(jnp.int32, (S, NB), 1)
    tri = (lax.broadcasted_iota(jnp.int32, (S, S), 0)
           >= lax.broadcasted_iota(jnp.int32, (S, S), 1)).astype(jnp.float32)
    up32 = (lax.broadcasted_iota(jnp.int32, (NB, NB), 0)
            < lax.broadcasted_iota(jnp.int32, (NB, NB), 1)).astype(jnp.float32)
    ident = (lax.broadcasted_iota(jnp.int32, (S, S), 0)
             == lax.broadcasted_iota(jnp.int32, (S, S), 1)).astype(jnp.float32)
    lane_iota = lax.broadcasted_iota(jnp.int32, (S, S), 1)
    iota_row = lax.broadcasted_iota(jnp.int32, (1, S), 1).astype(jnp.float32)

    st_rows, pos_rows = [], []
    for h in range(H):
        r = jnp.dot(qk, rot[:, h * HB:(h + 1) * HB],
                    preferred_element_type=jnp.float32)
        cat = jnp.concatenate([r, -r], axis=1)
        m = jnp.max(cat, axis=1, keepdims=True)
        bucket = jnp.min(jnp.where(cat == m, iota_nb, NB + 1),
                         axis=1, keepdims=True)
        oh = (bucket == iota_nb).astype(jnp.float32)
        csum = jnp.dot(tri, oh, preferred_element_type=jnp.float32)
        excl = csum - oh
        counts = jnp.sum(oh, axis=0, keepdims=True)
        offs = jnp.dot(counts, up32, preferred_element_type=jnp.float32,
                       precision=lax.Precision.HIGHEST)
        posf = jnp.sum(oh * (excl + offs), axis=1, keepdims=True)
        perm = (posf.astype(jnp.int32) == lane_iota).astype(jnp.float32)
        st_row = lax.dot_general(iota_row, perm, (((1,), (0,)), ((), ())),
                                 preferred_element_type=jnp.float32,
                                 precision=lax.Precision.HIGHEST)
        posT = lax.dot_general(posf, ident, (((0,), (0,)), ((), ())),
                               preferred_element_type=jnp.float32,
                               precision=lax.Precision.HIGHEST)
        st_rows.append(st_row)
        pos_rows.append(posT)
    st = jnp.concatenate(st_rows, axis=0).astype(jnp.int32)
    posT = jnp.concatenate(pos_rows, axis=0).astype(jnp.int32)
    stg_ref[0] = st + b * S
    h_iota = lax.broadcasted_iota(jnp.int32, (H, S), 0)
    posg_ref[0] = posT + (b * H + h_iota) * S


def _hash_pos_call(qk, v, rot2d, H):
    B, S, D = qk.shape
    return pl.pallas_call(
        _hash_pos_body,
        grid=(B,),
        in_specs=[
            pl.BlockSpec((1, S, D), lambda b: (b, 0, 0)),
            pl.BlockSpec((1, S, D), lambda b: (b, 0, 0)),
            pl.BlockSpec(rot2d.shape, lambda b: (0, 0)),
        ],
        out_specs=[
            pl.BlockSpec((1, H, S), lambda b: (b, 0, 0)),
            pl.BlockSpec((1, H, S), lambda b: (b, 0, 0)),
            pl.BlockSpec((1, S, 2 * D), lambda b: (b, 0, 0)),
        ],
        out_shape=[
            jax.ShapeDtypeStruct((B, H, S), jnp.int32),
            jax.ShapeDtypeStruct((B, H, S), jnp.int32),
            jax.ShapeDtypeStruct((B, S, 2 * D), jnp.float32),
        ],
    )(qk, v, rot2d)


def _attn_body(qvm_ref, qvp_ref, idsl_m_ref, idsl_p_ref, idss_ref, so_ref):
    CH = 16
    W = 64
    qvm = qvm_ref[0]
    qvp = qvp_ref[0]
    D = qvm.shape[1] // 2
    qm = qvm[:, 0:D]
    vm = qvm[:, D:]
    qp = qvp[:, 0:D]
    vp = qvp[:, D:]
    idsl_m = idsl_m_ref[0]
    idsl_p = idsl_p_ref[0]
    idss = idss_ref[0]

    k_shift = jnp.concatenate([qp[-W:, :], qm[:-W, :]], axis=0)
    v_shift = jnp.concatenate([vp[-W:, :], vm[:-W, :]], axis=0)
    ids_shift = jnp.concatenate([idsl_p[:, -W:], idsl_m[:, :-W]], axis=1)

    for c in range(CH):
        sl = slice(c * W, (c + 1) * W)
        q = qm[sl, :]
        kcat = jnp.concatenate([qm[sl, :], k_shift[sl, :]], axis=0)
        vcat = jnp.concatenate([vm[sl, :], v_shift[sl, :]], axis=0)
        norms = jnp.sqrt(jnp.sum(kcat * kcat, axis=1, keepdims=True))
        bk = kcat / jnp.maximum(norms, 1e-12)
        qt = idss[sl, :]
        kvt = jnp.concatenate([idsl_m[:, sl], ids_shift[:, sl]], axis=1)
        dots = lax.dot_general(q, bk, (((1,), (1,)), ((), ())),
                               preferred_element_type=jnp.float32) * 0.125
        dots = jnp.where(qt == kvt, -50000.0, dots)
        m = jnp.max(dots, axis=1, keepdims=True)
        e = jnp.exp(dots - m)
        s = jnp.sum(e, axis=1, keepdims=True)
        lse = m + jnp.log(s)
        p = e / s
        bo = jnp.dot(p, vcat, preferred_element_type=jnp.float32)
        so_ref[0, sl, 0:D] = bo
        so_ref[0, sl, D:D + 1] = lse
        so_ref[0, sl, D + 1:] = jnp.zeros((W, so_ref.shape[2] - D - 1),
                                          jnp.float32)


def _attn_call(sqv, stg):
    B, T, D2 = sqv.shape
    NBLK = 16
    BW = T // NBLK
    idsl = stg.reshape(B * NBLK, 1, BW)
    idss = stg.reshape(B, T, 1)

    def il_main(b, k):
        return (b * NBLK + k, 0, 0)

    def il_prev(b, k):
        return (b * NBLK + (k - 1) % NBLK, 0, 0)

    row_spec_m = pl.BlockSpec((1, BW, D2), lambda b, k: (b, k, 0))
    row_spec_p = pl.BlockSpec((1, BW, D2), lambda b, k: (b, (k - 1) % NBLK, 0))
    return pl.pallas_call(
        _attn_body,
        grid=(B, NBLK),
        in_specs=[
            row_spec_m,
            row_spec_p,
            pl.BlockSpec((1, 1, BW), il_main),
            pl.BlockSpec((1, 1, BW), il_prev),
            pl.BlockSpec((1, BW, 1), lambda b, k: (b, k, 0)),
        ],
        out_specs=pl.BlockSpec((1, BW, D2), lambda b, k: (b, k, 0)),
        out_shape=jax.ShapeDtypeStruct((B, T, D2), jnp.float32),
    )(sqv, sqv, idsl, idsl, idss)


def _combine_body(oe_ref, out_ref):
    oe = oe_ref[0]
    D = out_ref.shape[2]
    o = oe[:, :, 0:D]
    l = oe[:, :, D:D + 1]
    m = jnp.max(l, axis=0, keepdims=True)
    w = jnp.exp(l - m)
    s = jnp.sum(w, axis=0, keepdims=True)
    w = w / s
    out_ref[0] = jnp.sum(o * w, axis=0)


def _combine_call(oe_u, D):
    B, H, S, D2 = oe_u.shape
    TB = 512
    return pl.pallas_call(
        _combine_body,
        grid=(B, S // TB),
        in_specs=[
            pl.BlockSpec((1, H, TB, D2), lambda b, t: (b, 0, t, 0)),
        ],
        out_specs=pl.BlockSpec((1, TB, D), lambda b, t: (b, t, 0)),
        out_shape=jax.ShapeDtypeStruct((B, S, D), jnp.float32),
    )(oe_u)


def _sc_info():
    info = plsc.get_sparse_core_info()
    return info.num_cores, info.num_subcores, info.num_lanes


def _gather_rows_call(tables, idx):
    N = idx.shape[0]
    NC, NS, L = _sc_info()
    NW = NC * NS
    RPW = N // NW
    CHUNK = 128
    NCH = RPW // CHUNK
    NT = len(tables)

    @functools.partial(
        pl.kernel,
        mesh=plsc.VectorSubcoreMesh(core_axis_name="c", subcore_axis_name="s"),
        out_type=[jax.ShapeDtypeStruct((N, t.shape[1]), t.dtype)
                  for t in tables],
        scratch_types=(
            [pltpu.VMEM((CHUNK,), jnp.int32)]
            + [pltpu.VMEM((CHUNK, t.shape[1]), t.dtype) for t in tables]
            + [pltpu.SemaphoreType.DMA] * NT
        ),
    )
    def k(*refs):
        tabs = refs[:NT]
        idx_hbm = refs[NT]
        outs = refs[NT + 1:NT + 1 + NT]
        idx_v = refs[NT + 1 + NT]
        rows = refs[NT + 2 + NT:NT + 2 + 2 * NT]
        sems = refs[NT + 2 + 2 * NT:]
        wid = lax.axis_index("s") * NC + lax.axis_index("c")
        base = wid * RPW

        def body(c, _):
            start = base + c * CHUNK
            pltpu.sync_copy(idx_hbm.at[pl.ds(start, CHUNK)], idx_v)
            cps = [pltpu.async_copy(tabs[i].at[idx_v], rows[i], sems[i])
                   for i in range(NT)]
            for cp in cps:
                cp.wait()
            for i in range(NT):
                pltpu.sync_copy(rows[i], outs[i].at[pl.ds(start, CHUNK)])
            return 0

        lax.fori_loop(0, NCH, body, 0)

    return k(*tables, idx)


def kernel(qk, v, rotations):
    B, S, D = qk.shape
    H = rotations.shape[2]
    rot2d = rotations.reshape(D, H * rotations.shape[3])

    N = B * H * S
    stg, posg, qv = _hash_pos_call(qk, v, rot2d, H)
    (sqv,) = _gather_rows_call([qv.reshape(B * S, 2 * D)], stg.reshape(N))
    soe = _attn_call(sqv.reshape(B, H * S, 2 * D), stg.reshape(N))
    (oe_u,) = _gather_rows_call([soe.reshape(N, 2 * D)], posg.reshape(N))
    return _combine_call(oe_u.reshape(B, H, S, 2 * D), D)


# speedup vs baseline: 4.8665x; 1.8473x over previous
import functools

import jax
import jax.numpy as jnp
from jax import lax
from jax.experimental import pallas as pl
from jax.experimental.pallas import tpu as pltpu
from jax.experimental.pallas import tpu_sc as plsc


def _hash_pos_body(qk_ref, v_ref, rot_ref, posg_ref, qv_ref, bmask_ref):
    qk = qk_ref[0]
    rot = rot_ref[...]
    qv_ref[0] = jnp.concatenate([qk, v_ref[0]], axis=1)
    S = qk.shape[0]
    H = posg_ref.shape[1]
    HB = rot.shape[1] // H
    NB = 2 * HB
    W = 64
    b = pl.program_id(0)

    iota_nb = lax.broadcasted_iota(jnp.int32, (S, NB), 1)
    tri = (lax.broadcasted_iota(jnp.int32, (S, S), 0)
           >= lax.broadcasted_iota(jnp.int32, (S, S), 1)).astype(jnp.float32)
    up32 = (lax.broadcasted_iota(jnp.int32, (NB, NB), 0)
            < lax.broadcasted_iota(jnp.int32, (NB, NB), 1)).astype(jnp.float32)
    ident = (lax.broadcasted_iota(jnp.int32, (S, S), 0)
             == lax.broadcasted_iota(jnp.int32, (S, S), 1)).astype(jnp.float32)
    ident64 = (lax.broadcasted_iota(jnp.int32, (W, W), 0)
               == lax.broadcasted_iota(jnp.int32, (W, W), 1)).astype(jnp.float32)
    sub_iota = lax.broadcasted_iota(jnp.int32, (S, 1), 0)
    iota_s64 = lax.broadcasted_iota(jnp.int32, (S, W), 1)

    ohs = []
    for h in range(H):
        r = jnp.dot(qk, rot[:, h * HB:(h + 1) * HB],
                    preferred_element_type=jnp.float32)
        cat = jnp.concatenate([r, -r], axis=1)
        m = jnp.max(cat, axis=1, keepdims=True)
        bucket = jnp.min(jnp.where(cat == m, iota_nb, NB + 1),
                         axis=1, keepdims=True)
        ohs.append((bucket == iota_nb).astype(jnp.float32))
    oh8 = jnp.concatenate(ohs, axis=1)
    csum8 = jnp.dot(tri, oh8, preferred_element_type=jnp.float32)
    excl8 = csum8 - oh8
    counts8 = jnp.sum(oh8, axis=0, keepdims=True)

    hi_cols, lo_cols, brows = [], [], []
    for h in range(H):
        oh = ohs[h]
        excl = excl8[:, h * NB:(h + 1) * NB]
        counts = counts8[:, h * NB:(h + 1) * NB]
        offs = jnp.dot(counts, up32, preferred_element_type=jnp.float32,
                       precision=lax.Precision.HIGHEST)
        posf = jnp.sum(oh * (excl + offs), axis=1, keepdims=True)
        posi = posf.astype(jnp.int32)
        hi_cols.append((posi >> 8).astype(jnp.float32))
        lo_cols.append((posi & 255).astype(jnp.float32))
        first = (posi == iota_s64).astype(jnp.int32)
        last = (posi == iota_s64 + (S - W)).astype(jnp.int32)
        psub = jnp.concatenate([first, last], axis=1)
        bid = jnp.sum(psub * sub_iota, axis=0, keepdims=True)
        brows.append(bid)

    bm_list = []
    for h in range(H):
        qrow = brows[h][:, 0:W]
        prow = brows[(h - 1) % H][:, W:2 * W]
        qhi = (qrow >> 8).astype(jnp.float32)
        qlo = (qrow & 255).astype(jnp.float32)
        qhic = lax.dot_general(ident64, qhi, (((1,), (1,)), ((), ())),
                               preferred_element_type=jnp.float32)
        qloc = lax.dot_general(ident64, qlo, (((1,), (1,)), ((), ())),
                               preferred_element_type=jnp.float32)
        qcol = (256.0 * qhic + qloc).astype(jnp.int32)
        bm_list.append((qcol == prow).astype(jnp.int32))
    bmask_ref[0] = jnp.concatenate(bm_list, axis=0)

    X = jnp.concatenate(hi_cols + lo_cols, axis=1)
    XT = lax.dot_general(X, ident, (((0,), (0,)), ((), ())),
                         preferred_element_type=jnp.float32)
    posT8 = (256.0 * XT[0:H] + XT[H:2 * H]).astype(jnp.int32)
    h_iota = lax.broadcasted_iota(jnp.int32, (H, S), 0)
    posg_ref[0] = posT8 + (b * H + h_iota) * S


def _hash_pos_call(qk, v, rot2d, H):
    B, S, D = qk.shape
    W = 64
    return pl.pallas_call(
        _hash_pos_body,
        grid=(B,),
        in_specs=[
            pl.BlockSpec((1, S, D), lambda b: (b, 0, 0)),
            pl.BlockSpec((1, S, D), lambda b: (b, 0, 0)),
            pl.BlockSpec(rot2d.shape, lambda b: (0, 0)),
        ],
        out_specs=[
            pl.BlockSpec((1, H, S), lambda b: (b, 0, 0)),
            pl.BlockSpec((1, S, 2 * D), lambda b: (b, 0, 0)),
            pl.BlockSpec((1, H * W, W), lambda b: (b, 0, 0)),
        ],
        out_shape=[
            jax.ShapeDtypeStruct((B, H, S), jnp.int32),
            jax.ShapeDtypeStruct((B, S, 2 * D), jnp.float32),
            jax.ShapeDtypeStruct((B, H * W, W), jnp.int32),
        ],
    )(qk, v, rot2d)


def _attn_body(qvm_ref, qvp_ref, bm_ref, so_ref):
    CH = 16
    W = 64
    qvm = qvm_ref[0]
    qvp = qvp_ref[0]
    D = qvm.shape[1] // 2
    qm = qvm[:, 0:D]
    vm = qvm[:, D:]
    qp = qvp[:, 0:D]
    vp = qvp[:, D:]
    bm = bm_ref[0]

    k_shift = jnp.concatenate([qp[-W:, :], qm[:-W, :]], axis=0)
    v_shift = jnp.concatenate([vp[-W:, :], vm[:-W, :]], axis=0)

    diag = (lax.broadcasted_iota(jnp.int32, (W, W), 0)
            == lax.broadcasted_iota(jnp.int32, (W, W), 1)).astype(jnp.int32)
    is_even = 1 - (pl.program_id(1) % 2)
    bm0 = bm * is_even
    mask_c0 = jnp.concatenate([diag, bm0], axis=1)
    mask_rest = jnp.concatenate(
        [diag, jnp.zeros((W, W), jnp.int32)], axis=1)

    for c in range(CH):
        sl = slice(c * W, (c + 1) * W)
        q = qm[sl, :]
        kcat = jnp.concatenate([qm[sl, :], k_shift[sl, :]], axis=0)
        vcat = jnp.concatenate([vm[sl, :], v_shift[sl, :]], axis=0)
        norms = jnp.sqrt(jnp.sum(kcat * kcat, axis=1, keepdims=True))
        bk = kcat / jnp.maximum(norms, 1e-12)
        dots = lax.dot_general(q, bk, (((1,), (1,)), ((), ())),
                               preferred_element_type=jnp.float32) * 0.125
        mask = mask_c0 if c == 0 else mask_rest
        dots = jnp.where(mask > 0, -50000.0, dots)
        m = jnp.max(dots, axis=1, keepdims=True)
        e = jnp.exp(dots - m)
        s = jnp.sum(e, axis=1, keepdims=True)
        lse = m + jnp.log(s)
        p = e / s
        bo = jnp.dot(p, vcat, preferred_element_type=jnp.float32)
        so_ref[0, sl, 0:D] = bo
        so_ref[0, sl, D:D + 1] = lse
        so_ref[0, sl, D + 1:] = jnp.zeros((W, so_ref.shape[2] - D - 1),
                                          jnp.float32)


def _attn_call(sqv, bmask):
    B, T, D2 = sqv.shape
    NBLK = 16
    BW = T // NBLK
    W = 64

    row_spec_m = pl.BlockSpec((1, BW, D2), lambda b, k: (b, k, 0))
    row_spec_p = pl.BlockSpec((1, BW, D2), lambda b, k: (b, (k - 1) % NBLK, 0))
    return pl.pallas_call(
        _attn_body,
        grid=(B, NBLK),
        in_specs=[
            row_spec_m,
            row_spec_p,
            pl.BlockSpec((1, W, W), lambda b, k: (b, k // 2, 0)),
        ],
        out_specs=pl.BlockSpec((1, BW, D2), lambda b, k: (b, k, 0)),
        out_shape=jax.ShapeDtypeStruct((B, T, D2), jnp.float32),
    )(sqv, sqv, bmask)


def _combine_body(oe_ref, out_ref):
    oe = oe_ref[0]
    D = out_ref.shape[2]
    o = oe[:, :, 0:D]
    l = oe[:, :, D:D + 1]
    m = jnp.max(l, axis=0, keepdims=True)
    w = jnp.exp(l - m)
    s = jnp.sum(w, axis=0, keepdims=True)
    w = w / s
    out_ref[0] = jnp.sum(o * w, axis=0)


def _combine_call(oe_u, D):
    B, H, S, D2 = oe_u.shape
    TB = 512
    return pl.pallas_call(
        _combine_body,
        grid=(B, S // TB),
        in_specs=[
            pl.BlockSpec((1, H, TB, D2), lambda b, t: (b, 0, t, 0)),
        ],
        out_specs=pl.BlockSpec((1, TB, D), lambda b, t: (b, t, 0)),
        out_shape=jax.ShapeDtypeStruct((B, S, D), jnp.float32),
    )(oe_u)


def _sc_info():
    info = plsc.get_sparse_core_info()
    return info.num_cores, info.num_subcores, info.num_lanes


def _scatter_rows_call(qv_flat, posg_flat, B, S, H):
    N = B * H * S
    D2 = qv_flat.shape[1]
    NC, NS, L = _sc_info()
    NW = NC * NS
    PPW = (B * H) // NW
    CHUNK = 128
    NCH = S // CHUNK

    @functools.partial(
        pl.kernel,
        mesh=plsc.VectorSubcoreMesh(core_axis_name="c", subcore_axis_name="s"),
        out_type=jax.ShapeDtypeStruct((N, D2), jnp.float32),
        scratch_types=[
            pltpu.VMEM((CHUNK,), jnp.int32),
            pltpu.VMEM((CHUNK, D2), jnp.float32),
            pltpu.SemaphoreType.DMA,
        ],
    )
    def k(qv_hbm, posg_hbm, sqv_hbm, idx_v, rows_v, sem):
        wid = lax.axis_index("s") * NC + lax.axis_index("c")
        for j in range(PPW):
            bh = wid * PPW + j
            b = bh // H

            def body(c, _):
                tstart = c * CHUNK
                pltpu.sync_copy(posg_hbm.at[pl.ds(bh * S + tstart, CHUNK)],
                                idx_v)
                pltpu.sync_copy(qv_hbm.at[pl.ds(b * S + tstart, CHUNK)],
                                rows_v)
                pltpu.async_copy(rows_v, sqv_hbm.at[idx_v], sem).wait()
                return 0

            lax.fori_loop(0, NCH, body, 0)

    return k(qv_flat, posg_flat)


def _gather_rows_call(tables, idx):
    N = idx.shape[0]
    NC, NS, L = _sc_info()
    NW = NC * NS
    RPW = N // NW
    CHUNK = 128
    NCH = RPW // CHUNK
    NT = len(tables)

    @functools.partial(
        pl.kernel,
        mesh=plsc.VectorSubcoreMesh(core_axis_name="c", subcore_axis_name="s"),
        out_type=[jax.ShapeDtypeStruct((N, t.shape[1]), t.dtype)
                  for t in tables],
        scratch_types=(
            [pltpu.VMEM((CHUNK,), jnp.int32)]
            + [pltpu.VMEM((CHUNK, t.shape[1]), t.dtype) for t in tables]
            + [pltpu.SemaphoreType.DMA] * NT
        ),
    )
    def k(*refs):
        tabs = refs[:NT]
        idx_hbm = refs[NT]
        outs = refs[NT + 1:NT + 1 + NT]
        idx_v = refs[NT + 1 + NT]
        rows = refs[NT + 2 + NT:NT + 2 + 2 * NT]
        sems = refs[NT + 2 + 2 * NT:]
        wid = lax.axis_index("s") * NC + lax.axis_index("c")
        base = wid * RPW

        def body(c, _):
            start = base + c * CHUNK
            pltpu.sync_copy(idx_hbm.at[pl.ds(start, CHUNK)], idx_v)
            cps = [pltpu.async_copy(tabs[i].at[idx_v], rows[i], sems[i])
                   for i in range(NT)]
            for cp in cps:
                cp.wait()
            for i in range(NT):
                pltpu.sync_copy(rows[i], outs[i].at[pl.ds(start, CHUNK)])
            return 0

        lax.fori_loop(0, NCH, body, 0)

    return k(*tables, idx)


def kernel(qk, v, rotations):
    B, S, D = qk.shape
    H = rotations.shape[2]
    rot2d = rotations.reshape(D, H * rotations.shape[3])

    N = B * H * S
    posg, qv, bmask = _hash_pos_call(qk, v, rot2d, H)
    sqv = _scatter_rows_call(qv.reshape(B * S, 2 * D), posg.reshape(N),
                             B, S, H)
    soe = _attn_call(sqv.reshape(B, H * S, 2 * D), bmask)
    (oe_u,) = _gather_rows_call([soe.reshape(N, 2 * D)], posg.reshape(N))
    return _combine_call(oe_u.reshape(B, H, S, 2 * D), D)


# speedup vs baseline: 7.7971x; 1.6022x over previous
import functools

import jax
import jax.numpy as jnp
from jax import lax
from jax.experimental import pallas as pl
from jax.experimental.pallas import tpu as pltpu
from jax.experimental.pallas import tpu_sc as plsc


def _hash_pos_body(qk_ref, v_ref, rot_ref, posg_ref, qv_ref, bmask_ref):
    qk = qk_ref[0]
    rot = rot_ref[...]
    qv_ref[0] = jnp.concatenate([qk, v_ref[0]], axis=1)
    S = qk.shape[0]
    H = posg_ref.shape[1]
    HB = rot.shape[1] // H
    NB = 2 * HB
    W = 64
    b = pl.program_id(0)

    iota_nb = lax.broadcasted_iota(jnp.int32, (S, NB), 1)
    tri = (lax.broadcasted_iota(jnp.int32, (S, S), 0)
           >= lax.broadcasted_iota(jnp.int32, (S, S), 1)).astype(jnp.float32)
    up32 = (lax.broadcasted_iota(jnp.int32, (NB, NB), 0)
            < lax.broadcasted_iota(jnp.int32, (NB, NB), 1)).astype(jnp.float32)
    ident = (lax.broadcasted_iota(jnp.int32, (S, S), 0)
             == lax.broadcasted_iota(jnp.int32, (S, S), 1)).astype(jnp.float32)
    ident64 = (lax.broadcasted_iota(jnp.int32, (W, W), 0)
               == lax.broadcasted_iota(jnp.int32, (W, W), 1)).astype(jnp.float32)
    sub_iota = lax.broadcasted_iota(jnp.int32, (S, 1), 0)
    iota_s64 = lax.broadcasted_iota(jnp.int32, (S, W), 1)

    ohs = []
    for h in range(H):
        r = jnp.dot(qk, rot[:, h * HB:(h + 1) * HB],
                    preferred_element_type=jnp.float32)
        cat = jnp.concatenate([r, -r], axis=1)
        m = jnp.max(cat, axis=1, keepdims=True)
        bucket = jnp.min(jnp.where(cat == m, iota_nb, NB + 1),
                         axis=1, keepdims=True)
        ohs.append((bucket == iota_nb).astype(jnp.float32))
    oh8 = jnp.concatenate(ohs, axis=1)
    csum8 = jnp.dot(tri, oh8, preferred_element_type=jnp.float32)
    excl8 = csum8 - oh8
    counts8 = jnp.sum(oh8, axis=0, keepdims=True)

    hi_cols, lo_cols, brows = [], [], []
    for h in range(H):
        oh = ohs[h]
        excl = excl8[:, h * NB:(h + 1) * NB]
        counts = counts8[:, h * NB:(h + 1) * NB]
        offs = jnp.dot(counts, up32, preferred_element_type=jnp.float32,
                       precision=lax.Precision.HIGHEST)
        posf = jnp.sum(oh * (excl + offs), axis=1, keepdims=True)
        posi = posf.astype(jnp.int32)
        hi_cols.append((posi >> 8).astype(jnp.float32))
        lo_cols.append((posi & 255).astype(jnp.float32))
        first = (posi == iota_s64).astype(jnp.int32)
        last = (posi == iota_s64 + (S - W)).astype(jnp.int32)
        psub = jnp.concatenate([first, last], axis=1)
        bid = jnp.sum(psub * sub_iota, axis=0, keepdims=True)
        brows.append(bid)

    bm_list = []
    for h in range(H):
        qrow = brows[h][:, 0:W]
        prow = brows[(h - 1) % H][:, W:2 * W]
        qhi = (qrow >> 8).astype(jnp.float32)
        qlo = (qrow & 255).astype(jnp.float32)
        qhic = lax.dot_general(ident64, qhi, (((1,), (1,)), ((), ())),
                               preferred_element_type=jnp.float32)
        qloc = lax.dot_general(ident64, qlo, (((1,), (1,)), ((), ())),
                               preferred_element_type=jnp.float32)
        qcol = (256.0 * qhic + qloc).astype(jnp.int32)
        bm_list.append((qcol == prow).astype(jnp.int32))
    bmask_ref[0] = jnp.concatenate(bm_list, axis=0)

    X = jnp.concatenate(hi_cols + lo_cols, axis=1)
    XT = lax.dot_general(X, ident, (((0,), (0,)), ((), ())),
                         preferred_element_type=jnp.float32)
    posT8 = (256.0 * XT[0:H] + XT[H:2 * H]).astype(jnp.int32)
    h_iota = lax.broadcasted_iota(jnp.int32, (H, S), 0)
    posg_ref[0] = posT8 + (b * H + h_iota) * S


def _hash_pos_call(qk, v, rot2d, H):
    B, S, D = qk.shape
    W = 64
    return pl.pallas_call(
        _hash_pos_body,
        grid=(B,),
        in_specs=[
            pl.BlockSpec((1, S, D), lambda b: (b, 0, 0)),
            pl.BlockSpec((1, S, D), lambda b: (b, 0, 0)),
            pl.BlockSpec(rot2d.shape, lambda b: (0, 0)),
        ],
        out_specs=[
            pl.BlockSpec((1, H, S), lambda b: (b, 0, 0)),
            pl.BlockSpec((1, S, 2 * D), lambda b: (b, 0, 0)),
            pl.BlockSpec((1, H * W, W), lambda b: (b, 0, 0)),
        ],
        out_shape=[
            jax.ShapeDtypeStruct((B, H, S), jnp.int32),
            jax.ShapeDtypeStruct((B, S, 2 * D), jnp.float32),
            jax.ShapeDtypeStruct((B, H * W, W), jnp.int32),
        ],
    )(qk, v, rot2d)


def _attn_body(qvm_ref, qvp_ref, bm_ref, so_ref):
    CH = 16
    W = 64
    qvm = qvm_ref[0]
    qvp = qvp_ref[0]
    D = qvm.shape[1] // 2
    qm = qvm[:, 0:D]
    vm = qvm[:, D:]
    qp = qvp[:, 0:D]
    vp = qvp[:, D:]
    bm = bm_ref[0]

    norms = jnp.sqrt(jnp.sum(qm * qm, axis=1, keepdims=True))
    nm = qm / jnp.maximum(norms, 1e-12)
    qpl = qp[-W:, :]
    pnorm = jnp.sqrt(jnp.sum(qpl * qpl, axis=1, keepdims=True))
    npl = qpl / jnp.maximum(pnorm, 1e-12)
    n_shift = jnp.concatenate([npl, nm[:-W, :]], axis=0)
    v_shift = jnp.concatenate([vp[-W:, :], vm[:-W, :]], axis=0)

    q3 = qm.reshape(CH, W, D)
    n3 = nm.reshape(CH, W, D)
    ns3 = n_shift.reshape(CH, W, D)
    v3 = vm.reshape(CH, W, D)
    vs3 = v_shift.reshape(CH, W, D)

    bdims = (((2,), (2,)), ((0,), (0,)))
    dots_s = lax.dot_general(q3, n3, bdims,
                             preferred_element_type=jnp.float32) * 0.125
    dots_p = lax.dot_general(q3, ns3, bdims,
                             preferred_element_type=jnp.float32) * 0.125

    diag3 = (lax.broadcasted_iota(jnp.int32, (CH, W, W), 1)
             == lax.broadcasted_iota(jnp.int32, (CH, W, W), 2)).astype(jnp.int32)
    is_even = 1 - (pl.program_id(1) % 2)
    c_is0 = (lax.broadcasted_iota(jnp.int32, (CH, W, W), 0) == 0).astype(jnp.int32)
    bm3 = bm[None, :, :] * (c_is0 * is_even)
    dots_s = jnp.where(diag3 > 0, -50000.0, dots_s)
    dots_p = jnp.where(bm3 > 0, -50000.0, dots_p)

    m = jnp.maximum(jnp.max(dots_s, axis=2, keepdims=True),
                    jnp.max(dots_p, axis=2, keepdims=True))
    e_s = jnp.exp(dots_s - m)
    e_p = jnp.exp(dots_p - m)
    s = jnp.sum(e_s, axis=2, keepdims=True) + jnp.sum(e_p, axis=2, keepdims=True)
    lse = m + jnp.log(s)
    pdims = (((2,), (1,)), ((0,), (0,)))
    bo = (lax.dot_general(e_s / s, v3, pdims,
                          preferred_element_type=jnp.float32)
          + lax.dot_general(e_p / s, vs3, pdims,
                            preferred_element_type=jnp.float32))
    so_ref[0, :, 0:D] = bo.reshape(CH * W, D)
    so_ref[0, :, D:D + 1] = lse.reshape(CH * W, 1)
    so_ref[0, :, D + 1:] = jnp.zeros((CH * W, so_ref.shape[2] - D - 1),
                                     jnp.float32)


def _attn_call(sqv, bmask):
    B, T, D2 = sqv.shape
    NBLK = 16
    BW = T // NBLK
    W = 64

    row_spec_m = pl.BlockSpec((1, BW, D2), lambda b, k: (b, k, 0))
    row_spec_p = pl.BlockSpec((1, BW, D2), lambda b, k: (b, (k - 1) % NBLK, 0))
    return pl.pallas_call(
        _attn_body,
        grid=(B, NBLK),
        in_specs=[
            row_spec_m,
            row_spec_p,
            pl.BlockSpec((1, W, W), lambda b, k: (b, k // 2, 0)),
        ],
        out_specs=pl.BlockSpec((1, BW, D2), lambda b, k: (b, k, 0)),
        out_shape=jax.ShapeDtypeStruct((B, T, D2), jnp.float32),
    )(sqv, sqv, bmask)


def _combine_body(oe_ref, out_ref):
    oe = oe_ref[0]
    D = out_ref.shape[2]
    o = oe[:, :, 0:D]
    l = oe[:, :, D:D + 1]
    m = jnp.max(l, axis=0, keepdims=True)
    w = jnp.exp(l - m)
    s = jnp.sum(w, axis=0, keepdims=True)
    w = w / s
    out_ref[0] = jnp.sum(o * w, axis=0)


def _combine_call(oe_u, D):
    B, H, S, D2 = oe_u.shape
    TB = 512
    return pl.pallas_call(
        _combine_body,
        grid=(B, S // TB),
        in_specs=[
            pl.BlockSpec((1, H, TB, D2), lambda b, t: (b, 0, t, 0)),
        ],
        out_specs=pl.BlockSpec((1, TB, D), lambda b, t: (b, t, 0)),
        out_shape=jax.ShapeDtypeStruct((B, S, D), jnp.float32),
    )(oe_u)


def _sc_info():
    info = plsc.get_sparse_core_info()
    return info.num_cores, info.num_subcores, info.num_lanes


def _scatter_rows_call(qv_flat, posg_flat, B, S, H):
    N = B * H * S
    D2 = qv_flat.shape[1]
    NC, NS, L = _sc_info()
    NW = NC * NS
    PPW = (B * H) // NW
    CHUNK = 128
    NCH = S // CHUNK

    @functools.partial(
        pl.kernel,
        mesh=plsc.VectorSubcoreMesh(core_axis_name="c", subcore_axis_name="s"),
        out_type=jax.ShapeDtypeStruct((N, D2), jnp.float32),
        scratch_types=[
            pltpu.VMEM((CHUNK,), jnp.int32),
            pltpu.VMEM((CHUNK, D2), jnp.float32),
            pltpu.SemaphoreType.DMA,
        ],
    )
    def k(qv_hbm, posg_hbm, sqv_hbm, idx_v, rows_v, sem):
        wid = lax.axis_index("s") * NC + lax.axis_index("c")
        for j in range(PPW):
            bh = wid * PPW + j
            b = bh // H

            def body(c, _):
                tstart = c * CHUNK
                pltpu.sync_copy(posg_hbm.at[pl.ds(bh * S + tstart, CHUNK)],
                                idx_v)
                pltpu.sync_copy(qv_hbm.at[pl.ds(b * S + tstart, CHUNK)],
                                rows_v)
                pltpu.async_copy(rows_v, sqv_hbm.at[idx_v], sem).wait()
                return 0

            lax.fori_loop(0, NCH, body, 0)

    return k(qv_flat, posg_flat)


def _gather_rows_call(tables, idx):
    N = idx.shape[0]
    NC, NS, L = _sc_info()
    NW = NC * NS
    RPW = N // NW
    CHUNK = 128
    NCH = RPW // CHUNK
    NT = len(tables)

    @functools.partial(
        pl.kernel,
        mesh=plsc.VectorSubcoreMesh(core_axis_name="c", subcore_axis_name="s"),
        out_type=[jax.ShapeDtypeStruct((N, t.shape[1]), t.dtype)
                  for t in tables],
        scratch_types=(
            [pltpu.VMEM((CHUNK,), jnp.int32)]
            + [pltpu.VMEM((CHUNK, t.shape[1]), t.dtype) for t in tables]
            + [pltpu.SemaphoreType.DMA] * NT
        ),
    )
    def k(*refs):
        tabs = refs[:NT]
        idx_hbm = refs[NT]
        outs = refs[NT + 1:NT + 1 + NT]
        idx_v = refs[NT + 1 + NT]
        rows = refs[NT + 2 + NT:NT + 2 + 2 * NT]
        sems = refs[NT + 2 + 2 * NT:]
        wid = lax.axis_index("s") * NC + lax.axis_index("c")
        base = wid * RPW

        def body(c, _):
            start = base + c * CHUNK
            pltpu.sync_copy(idx_hbm.at[pl.ds(start, CHUNK)], idx_v)
            cps = [pltpu.async_copy(tabs[i].at[idx_v], rows[i], sems[i])
                   for i in range(NT)]
            for cp in cps:
                cp.wait()
            for i in range(NT):
                pltpu.sync_copy(rows[i], outs[i].at[pl.ds(start, CHUNK)])
            return 0

        lax.fori_loop(0, NCH, body, 0)

    return k(*tables, idx)


def kernel(qk, v, rotations):
    B, S, D = qk.shape
    H = rotations.shape[2]
    rot2d = rotations.reshape(D, H * rotations.shape[3])

    N = B * H * S
    posg, qv, bmask = _hash_pos_call(qk, v, rot2d, H)
    sqv = _scatter_rows_call(qv.reshape(B * S, 2 * D), posg.reshape(N),
                             B, S, H)
    soe = _attn_call(sqv.reshape(B, H * S, 2 * D), bmask)
    (oe_u,) = _gather_rows_call([soe.reshape(N, 2 * D)], posg.reshape(N))
    return _combine_call(oe_u.reshape(B, H, S, 2 * D), D)


# speedup vs baseline: 7.9705x; 1.0222x over previous
import functools

import jax
import jax.numpy as jnp
from jax import lax
from jax.experimental import pallas as pl
from jax.experimental.pallas import tpu as pltpu
from jax.experimental.pallas import tpu_sc as plsc


def _hash_pos_body(qk_ref, v_ref, rot_ref, posg_ref, qv_ref, bmask_ref):
    qk = qk_ref[0]
    rot = rot_ref[...]
    qv_ref[0] = jnp.concatenate([qk, v_ref[0]], axis=1)
    S = qk.shape[0]
    H = posg_ref.shape[1]
    HB = rot.shape[1] // H
    NB = 2 * HB
    W = 64
    b = pl.program_id(0)

    iota_nb = lax.broadcasted_iota(jnp.int32, (S, NB), 1)
    up32 = (lax.broadcasted_iota(jnp.int32, (NB, NB), 0)
            < lax.broadcasted_iota(jnp.int32, (NB, NB), 1)).astype(jnp.float32)
    ident = (lax.broadcasted_iota(jnp.int32, (S, S), 0)
             == lax.broadcasted_iota(jnp.int32, (S, S), 1)).astype(jnp.float32)
    ident64 = (lax.broadcasted_iota(jnp.int32, (W, W), 0)
               == lax.broadcasted_iota(jnp.int32, (W, W), 1)).astype(jnp.float32)
    sub_iota = lax.broadcasted_iota(jnp.int32, (S, 1), 0)
    iota_s64 = lax.broadcasted_iota(jnp.int32, (S, W), 1)

    ohs = []
    for h in range(H):
        r = jnp.dot(qk, rot[:, h * HB:(h + 1) * HB],
                    preferred_element_type=jnp.float32)
        cat = jnp.concatenate([r, -r], axis=1)
        m = jnp.max(cat, axis=1, keepdims=True)
        bucket = jnp.min(jnp.where(cat == m, iota_nb, NB + 1),
                         axis=1, keepdims=True)
        ohs.append((bucket == iota_nb).astype(jnp.float32))
    oh8 = jnp.concatenate(ohs, axis=1)
    BLK = 256
    NBK = S // BLK
    tri_blk = (lax.broadcasted_iota(jnp.int32, (BLK, BLK), 0)
               >= lax.broadcasted_iota(jnp.int32, (BLK, BLK), 1)
               ).astype(jnp.float32)
    l8 = (lax.broadcasted_iota(jnp.int32, (NBK, NBK), 0)
          > lax.broadcasted_iota(jnp.int32, (NBK, NBK), 1)).astype(jnp.float32)
    csums = [jnp.dot(tri_blk, oh8[k * BLK:(k + 1) * BLK, :],
                     preferred_element_type=jnp.float32) for k in range(NBK)]
    bs = jnp.concatenate([c[BLK - 1:BLK, :] for c in csums], axis=0)
    carry = jnp.dot(l8, bs, preferred_element_type=jnp.float32,
                    precision=lax.Precision.HIGHEST)
    csum8 = jnp.concatenate(
        [csums[k] + carry[k:k + 1, :] for k in range(NBK)], axis=0)
    excl8 = csum8 - oh8
    counts8 = jnp.sum(oh8, axis=0, keepdims=True)

    hi_cols, lo_cols, brows = [], [], []
    for h in range(H):
        oh = ohs[h]
        excl = excl8[:, h * NB:(h + 1) * NB]
        counts = counts8[:, h * NB:(h + 1) * NB]
        offs = jnp.dot(counts, up32, preferred_element_type=jnp.float32,
                       precision=lax.Precision.HIGHEST)
        posf = jnp.sum(oh * (excl + offs), axis=1, keepdims=True)
        posi = posf.astype(jnp.int32)
        hi_cols.append((posi >> 8).astype(jnp.float32))
        lo_cols.append((posi & 255).astype(jnp.float32))
        first = (posi == iota_s64).astype(jnp.int32)
        last = (posi == iota_s64 + (S - W)).astype(jnp.int32)
        psub = jnp.concatenate([first, last], axis=1)
        bid = jnp.sum(psub * sub_iota, axis=0, keepdims=True)
        brows.append(bid)

    bm_list = []
    for h in range(H):
        qrow = brows[h][:, 0:W]
        prow = brows[(h - 1) % H][:, W:2 * W]
        qhi = (qrow >> 8).astype(jnp.float32)
        qlo = (qrow & 255).astype(jnp.float32)
        qhic = lax.dot_general(ident64, qhi, (((1,), (1,)), ((), ())),
                               preferred_element_type=jnp.float32)
        qloc = lax.dot_general(ident64, qlo, (((1,), (1,)), ((), ())),
                               preferred_element_type=jnp.float32)
        qcol = (256.0 * qhic + qloc).astype(jnp.int32)
        bm_list.append((qcol == prow).astype(jnp.int32))
    bmask_ref[0] = jnp.concatenate(bm_list, axis=0)

    X = jnp.concatenate(hi_cols + lo_cols, axis=1)
    XT = lax.dot_general(X, ident, (((0,), (0,)), ((), ())),
                         preferred_element_type=jnp.float32)
    posT8 = (256.0 * XT[0:H] + XT[H:2 * H]).astype(jnp.int32)
    h_iota = lax.broadcasted_iota(jnp.int32, (H, S), 0)
    posg_ref[0] = posT8 + (b * H + h_iota) * S


def _hash_pos_call(qk, v, rot2d, H):
    B, S, D = qk.shape
    W = 64
    return pl.pallas_call(
        _hash_pos_body,
        grid=(B,),
        in_specs=[
            pl.BlockSpec((1, S, D), lambda b: (b, 0, 0)),
            pl.BlockSpec((1, S, D), lambda b: (b, 0, 0)),
            pl.BlockSpec(rot2d.shape, lambda b: (0, 0)),
        ],
        out_specs=[
            pl.BlockSpec((1, H, S), lambda b: (b, 0, 0)),
            pl.BlockSpec((1, S, 2 * D), lambda b: (b, 0, 0)),
            pl.BlockSpec((1, H * W, W), lambda b: (b, 0, 0)),
        ],
        out_shape=[
            jax.ShapeDtypeStruct((B, H, S), jnp.int32),
            jax.ShapeDtypeStruct((B, S, 2 * D), jnp.float32),
            jax.ShapeDtypeStruct((B, H * W, W), jnp.int32),
        ],
    )(qk, v, rot2d)


def _attn_body(qvm_ref, qvp_ref, bm_ref, so_ref):
    CH = 16
    W = 64
    qvm = qvm_ref[0]
    qvp = qvp_ref[0]
    D = qvm.shape[1] // 2
    qm = qvm[:, 0:D]
    vm = qvm[:, D:]
    qp = qvp[:, 0:D]
    vp = qvp[:, D:]
    bm = bm_ref[0]

    norms = jnp.sqrt(jnp.sum(qm * qm, axis=1, keepdims=True))
    nm = qm / jnp.maximum(norms, 1e-12)
    qpl = qp[-W:, :]
    pnorm = jnp.sqrt(jnp.sum(qpl * qpl, axis=1, keepdims=True))
    npl = qpl / jnp.maximum(pnorm, 1e-12)
    n_shift = jnp.concatenate([npl, nm[:-W, :]], axis=0)
    v_shift = jnp.concatenate([vp[-W:, :], vm[:-W, :]], axis=0)

    q3 = qm.reshape(CH, W, D)
    n3 = nm.reshape(CH, W, D)
    ns3 = n_shift.reshape(CH, W, D)
    v3 = vm.reshape(CH, W, D)
    vs3 = v_shift.reshape(CH, W, D)

    bdims = (((2,), (2,)), ((0,), (0,)))
    dots_s = lax.dot_general(q3, n3, bdims,
                             preferred_element_type=jnp.float32) * 0.125
    dots_p = lax.dot_general(q3, ns3, bdims,
                             preferred_element_type=jnp.float32) * 0.125

    diag3 = (lax.broadcasted_iota(jnp.int32, (CH, W, W), 1)
             == lax.broadcasted_iota(jnp.int32, (CH, W, W), 2)).astype(jnp.int32)
    is_even = 1 - (pl.program_id(1) % 2)
    c_is0 = (lax.broadcasted_iota(jnp.int32, (CH, W, W), 0) == 0).astype(jnp.int32)
    bm3 = bm[None, :, :] * (c_is0 * is_even)
    dots_s = jnp.where(diag3 > 0, -50000.0, dots_s)
    dots_p = jnp.where(bm3 > 0, -50000.0, dots_p)

    m = jnp.maximum(jnp.max(dots_s, axis=2, keepdims=True),
                    jnp.max(dots_p, axis=2, keepdims=True))
    e_s = jnp.exp(dots_s - m)
    e_p = jnp.exp(dots_p - m)
    s = jnp.sum(e_s, axis=2, keepdims=True) + jnp.sum(e_p, axis=2, keepdims=True)
    lse = m + jnp.log(s)
    pdims = (((2,), (1,)), ((0,), (0,)))
    bo = (lax.dot_general(e_s / s, v3, pdims,
                          preferred_element_type=jnp.float32)
          + lax.dot_general(e_p / s, vs3, pdims,
                            preferred_element_type=jnp.float32))
    so_ref[0, :, 0:D] = bo.reshape(CH * W, D)
    so_ref[0, :, D:D + 1] = lse.reshape(CH * W, 1)
    so_ref[0, :, D + 1:] = jnp.zeros((CH * W, so_ref.shape[2] - D - 1),
                                     jnp.float32)


def _attn_call(sqv, bmask):
    B, T, D2 = sqv.shape
    NBLK = 16
    BW = T // NBLK
    W = 64

    row_spec_m = pl.BlockSpec((1, BW, D2), lambda b, k: (b, k, 0))
    row_spec_p = pl.BlockSpec((1, BW, D2), lambda b, k: (b, (k - 1) % NBLK, 0))
    return pl.pallas_call(
        _attn_body,
        grid=(B, NBLK),
        in_specs=[
            row_spec_m,
            row_spec_p,
            pl.BlockSpec((1, W, W), lambda b, k: (b, k // 2, 0)),
        ],
        out_specs=pl.BlockSpec((1, BW, D2), lambda b, k: (b, k, 0)),
        out_shape=jax.ShapeDtypeStruct((B, T, D2), jnp.float32),
    )(sqv, sqv, bmask)


def _combine_body(oe_ref, out_ref):
    oe = oe_ref[0]
    D = out_ref.shape[2]
    o = oe[:, :, 0:D]
    l = oe[:, :, D:D + 1]
    m = jnp.max(l, axis=0, keepdims=True)
    w = jnp.exp(l - m)
    s = jnp.sum(w, axis=0, keepdims=True)
    w = w / s
    out_ref[0] = jnp.sum(o * w, axis=0)


def _combine_call(oe_u, D):
    B, H, S, D2 = oe_u.shape
    TB = 512
    return pl.pallas_call(
        _combine_body,
        grid=(B, S // TB),
        in_specs=[
            pl.BlockSpec((1, H, TB, D2), lambda b, t: (b, 0, t, 0)),
        ],
        out_specs=pl.BlockSpec((1, TB, D), lambda b, t: (b, t, 0)),
        out_shape=jax.ShapeDtypeStruct((B, S, D), jnp.float32),
    )(oe_u)


def _sc_info():
    info = plsc.get_sparse_core_info()
    return info.num_cores, info.num_subcores, info.num_lanes


def _scatter_rows_call(qv_flat, posg_flat, B, S, H):
    N = B * H * S
    D2 = qv_flat.shape[1]
    NC, NS, L = _sc_info()
    NW = NC * NS
    PPW = (B * H) // NW
    CHUNK = 128
    NCH = S // CHUNK

    @functools.partial(
        pl.kernel,
        mesh=plsc.VectorSubcoreMesh(core_axis_name="c", subcore_axis_name="s"),
        out_type=jax.ShapeDtypeStruct((N, D2), jnp.float32),
        scratch_types=[
            pltpu.VMEM((CHUNK,), jnp.int32),
            pltpu.VMEM((CHUNK, D2), jnp.float32),
            pltpu.SemaphoreType.DMA,
        ],
    )
    def k(qv_hbm, posg_hbm, sqv_hbm, idx_v, rows_v, sem):
        wid = lax.axis_index("s") * NC + lax.axis_index("c")
        for j in range(PPW):
            bh = wid * PPW + j
            b = bh // H

            def body(c, _):
                tstart = c * CHUNK
                pltpu.sync_copy(posg_hbm.at[pl.ds(bh * S + tstart, CHUNK)],
                                idx_v)
                pltpu.sync_copy(qv_hbm.at[pl.ds(b * S + tstart, CHUNK)],
                                rows_v)
                pltpu.async_copy(rows_v, sqv_hbm.at[idx_v], sem).wait()
                return 0

            lax.fori_loop(0, NCH, body, 0)

    return k(qv_flat, posg_flat)


def _gather_rows_call(tables, idx):
    N = idx.shape[0]
    NC, NS, L = _sc_info()
    NW = NC * NS
    RPW = N // NW
    CHUNK = 128
    NCH = RPW // CHUNK
    NT = len(tables)

    @functools.partial(
        pl.kernel,
        mesh=plsc.VectorSubcoreMesh(core_axis_name="c", subcore_axis_name="s"),
        out_type=[jax.ShapeDtypeStruct((N, t.shape[1]), t.dtype)
                  for t in tables],
        scratch_types=(
            [pltpu.VMEM((CHUNK,), jnp.int32)]
            + [pltpu.VMEM((CHUNK, t.shape[1]), t.dtype) for t in tables]
            + [pltpu.SemaphoreType.DMA] * NT
        ),
    )
    def k(*refs):
        tabs = refs[:NT]
        idx_hbm = refs[NT]
        outs = refs[NT + 1:NT + 1 + NT]
        idx_v = refs[NT + 1 + NT]
        rows = refs[NT + 2 + NT:NT + 2 + 2 * NT]
        sems = refs[NT + 2 + 2 * NT:]
        wid = lax.axis_index("s") * NC + lax.axis_index("c")
        base = wid * RPW

        def body(c, _):
            start = base + c * CHUNK
            pltpu.sync_copy(idx_hbm.at[pl.ds(start, CHUNK)], idx_v)
            cps = [pltpu.async_copy(tabs[i].at[idx_v], rows[i], sems[i])
                   for i in range(NT)]
            for cp in cps:
                cp.wait()
            for i in range(NT):
                pltpu.sync_copy(rows[i], outs[i].at[pl.ds(start, CHUNK)])
            return 0

        lax.fori_loop(0, NCH, body, 0)

    return k(*tables, idx)


def kernel(qk, v, rotations):
    B, S, D = qk.shape
    H = rotations.shape[2]
    rot2d = rotations.reshape(D, H * rotations.shape[3])

    N = B * H * S
    posg, qv, bmask = _hash_pos_call(qk, v, rot2d, H)
    sqv = _scatter_rows_call(qv.reshape(B * S, 2 * D), posg.reshape(N),
                             B, S, H)
    soe = _attn_call(sqv.reshape(B, H * S, 2 * D), bmask)
    (oe_u,) = _gather_rows_call([soe.reshape(N, 2 * D)], posg.reshape(N))
    return _combine_call(oe_u.reshape(B, H, S, 2 * D), D)
